# baseline (speedup 1.0000x reference)
import functools

import jax
import jax.numpy as jnp
from jax import lax
from jax.experimental import pallas as pl
from jax.experimental.pallas import tpu as pltpu
from jax.experimental.pallas import tpu_sc as plsc

N = 10000
E = 160000
EPAD = 163840
ROWS = EPAD // 128
NACC = 10016
NC, NS = 2, 16
NW = NC * NS
RPW = ROWS // NW
LEAK = 0.1
EPS = 1e-5
BLK = 1024


def _leaky(v):
    return jnp.where(v >= 0, v, LEAK * v)


def _dot(a, b):
    return lax.dot_general(a, b, (((1,), (0,)), ((), ())),
                           preferred_element_type=jnp.float32,
                           precision=lax.Precision.HIGHEST)


def _prep(x, et, gx, bx, ge, be):
    def body(x_ref, et_ref, gx_ref, bx_ref, ge_ref, be_ref,
             xo_ref, se_ref, te_ref):
        xv = x_ref[...]
        m = jnp.mean(xv, axis=0, keepdims=True)
        v = jnp.mean((xv - m) ** 2, axis=0, keepdims=True)
        xo_ref[...] = (xv - m) * lax.rsqrt(v + EPS) * gx_ref[...] + bx_ref[...]
        ev = et_ref[...]
        me = jnp.mean(ev, axis=1, keepdims=True)
        ve = jnp.mean((ev - me) ** 2, axis=1, keepdims=True)
        s = lax.rsqrt(ve + EPS) * ge_ref[...]
        se_ref[...] = s
        te_ref[...] = be_ref[...] - me * s

    return pl.pallas_call(
        body,
        out_shape=(jax.ShapeDtypeStruct((N, 16), jnp.float32),
                   jax.ShapeDtypeStruct((10, 1), jnp.float32),
                   jax.ShapeDtypeStruct((10, 1), jnp.float32)),
    )(x, et, gx, bx, ge, be)


def _sc_gather(table, idx2, m_rows, d):
    rpw = m_rows // NW
    nsup = rpw // 8
    mesh = plsc.VectorSubcoreMesh(core_axis_name="c", subcore_axis_name="s")

    @functools.partial(
        pl.kernel,
        out_type=jax.ShapeDtypeStruct((m_rows * 128, d), jnp.float32),
        mesh=mesh,
        compiler_params=pltpu.CompilerParams(use_tc_tiling_on_sc=False),
        scratch_types=[pltpu.VMEM((8, 128), jnp.int32),
                       pltpu.VMEM((128, d), jnp.float32)],
    )
    def k(table_hbm, idx_hbm, out_hbm, idx_v, rows_v):
        c = lax.axis_index("c")
        s = lax.axis_index("s")
        row0 = (s * NC + c) * rpw

        @pl.loop(0, nsup)
        def _sup(jb):
            r0 = row0 + jb * 8
            pltpu.sync_copy(idx_hbm.at[pl.ds(r0, 8)], idx_v)

            @pl.loop(0, 8)
            def _row(r):
                pltpu.sync_copy(table_hbm.at[idx_v.at[r]], rows_v)
                pltpu.sync_copy(rows_v, out_hbm.at[pl.ds((r0 + r) * 128, 128)])

    return k(table, idx2)


def _sc_scatter(msg, dst2, zeros, d):
    mesh = plsc.VectorSubcoreMesh(core_axis_name="c", subcore_axis_name="s")
    rps = NACC // NS

    @functools.partial(
        pl.kernel,
        out_type=jax.ShapeDtypeStruct((NC, NACC, d), jnp.float32),
        mesh=mesh,
        compiler_params=pltpu.CompilerParams(use_tc_tiling_on_sc=False),
        scratch_types=[pltpu.VMEM((8, 128), jnp.int32),
                       pltpu.VMEM((128, d), jnp.float32),
                       pltpu.VMEM_SHARED((NACC, d), jnp.float32)],
    )
    def k(msg_hbm, dst_hbm, z_hbm, out_hbm, idx_v, rows_v, acc_sh):
        c = lax.axis_index("c")
        s = lax.axis_index("s")
        row0 = (s * NC + c) * RPW
        rs = s * rps
        pltpu.sync_copy(z_hbm.at[pl.ds(rs, rps)], acc_sh.at[pl.ds(rs, rps)])
        plsc.subcore_barrier()

        @pl.loop(0, RPW // 8)
        def _sup(jb):
            r0 = row0 + jb * 8
            pltpu.sync_copy(dst_hbm.at[pl.ds(r0, 8)], idx_v)

            @pl.loop(0, 8)
            def _row(r):
                pltpu.sync_copy(msg_hbm.at[pl.ds((r0 + r) * 128, 128)], rows_v)
                pltpu.sync_copy(rows_v, acc_sh.at[idx_v.at[r]], add=True)

        plsc.subcore_barrier()
        pltpu.sync_copy(acc_sh.at[pl.ds(rs, rps)], out_hbm.at[c, pl.ds(rs, rps)])

    return k(msg, dst2, zeros)


def _msg(e_pad, se, te, xg, w1, b1, w2, b2, rm, fm, nin, nout):
    g = EPAD // BLK

    def body(e_ref, se_ref, te_ref, xg_ref, w1_ref, b1_ref, w2_ref, b2_ref,
             r_ref, f_ref, o_ref):
        eb = e_ref[...] * se_ref[...] + te_ref[...]
        h1 = _leaky(_dot(eb, w1_ref[...]) + b1_ref[...])
        w = _leaky(_dot(h1, w2_ref[...]) + b2_ref[...])
        rep = _dot(xg_ref[...], r_ref[...])
        o_ref[...] = _dot(w * rep, f_ref[...])

    k = nin * nout
    return pl.pallas_call(
        body,
        grid=(g,),
        in_specs=[pl.BlockSpec((BLK, 10), lambda i: (i, 0)),
                  pl.BlockSpec((1, 10), lambda i: (0, 0)),
                  pl.BlockSpec((1, 10), lambda i: (0, 0)),
                  pl.BlockSpec((BLK, nin), lambda i: (i, 0)),
                  pl.BlockSpec((10, nin), lambda i: (0, 0)),
                  pl.BlockSpec((1, nin), lambda i: (0, 0)),
                  pl.BlockSpec((nin, k), lambda i: (0, 0)),
                  pl.BlockSpec((1, k), lambda i: (0, 0)),
                  pl.BlockSpec((nin, k), lambda i: (0, 0)),
                  pl.BlockSpec((k, nout), lambda i: (0, 0))],
        out_specs=pl.BlockSpec((BLK, nout), lambda i: (i, 0)),
        out_shape=jax.ShapeDtypeStruct((EPAD, nout), jnp.float32),
    )(e_pad, se, te, xg, w1, b1, w2, b2, rm, fm)


def _update(part, xin, root, bias):
    nout = root.shape[1]

    def body(p_ref, x_ref, r_ref, b_ref, o_ref):
        o_ref[...] = (p_ref[0] + p_ref[1] + _dot(x_ref[...], r_ref[...])
                      + b_ref[...])

    return pl.pallas_call(
        body,
        out_shape=jax.ShapeDtypeStruct((N, nout), jnp.float32),
    )(part, xin, root, bias)


def _final(xf, e_pad, se, te, wa, wb, wc, b1, w2, b2, w3, b3, w4, b4, w5, b5):
    g = EPAD // BLK

    def body(xs_ref, xd_ref, e_ref, se_ref, te_ref, wa_ref, wb_ref, wc_ref,
             b1_ref, w2_ref, b2_ref, w3_ref, b3_ref, w4_ref, b4_ref,
             w5_ref, b5_ref, o_ref):
        eb = e_ref[...] * se_ref[...] + te_ref[...]
        h = _leaky(_dot(xs_ref[...], wa_ref[...]) + _dot(xd_ref[...], wb_ref[...])
                   + _dot(eb, wc_ref[...]) + b1_ref[...])
        h = _leaky(_dot(h, w2_ref[...]) + b2_ref[...])
        h = _leaky(_dot(h, w3_ref[...]) + b3_ref[...])
        h = _leaky(_dot(h, w4_ref[...]) + b4_ref[...])
        o_ref[...] = _dot(h, w5_ref[...]) + b5_ref[...]

    return pl.pallas_call(
        body,
        grid=(g,),
        in_specs=[pl.BlockSpec((BLK, 32), lambda i: (i, 0)),
                  pl.BlockSpec((BLK, 32), lambda i: (i + g, 0)),
                  pl.BlockSpec((BLK, 10), lambda i: (i, 0)),
                  pl.BlockSpec((1, 10), lambda i: (0, 0)),
                  pl.BlockSpec((1, 10), lambda i: (0, 0)),
                  pl.BlockSpec((32, 64), lambda i: (0, 0)),
                  pl.BlockSpec((32, 64), lambda i: (0, 0)),
                  pl.BlockSpec((10, 64), lambda i: (0, 0)),
                  pl.BlockSpec((1, 64), lambda i: (0, 0)),
                  pl.BlockSpec((64, 32), lambda i: (0, 0)),
                  pl.BlockSpec((1, 32), lambda i: (0, 0)),
                  pl.BlockSpec((32, 16), lambda i: (0, 0)),
                  pl.BlockSpec((1, 16), lambda i: (0, 0)),
                  pl.BlockSpec((16, 8), lambda i: (0, 0)),
                  pl.BlockSpec((1, 8), lambda i: (0, 0)),
                  pl.BlockSpec((8, 2), lambda i: (0, 0)),
                  pl.BlockSpec((1, 2), lambda i: (0, 0))],
        out_specs=pl.BlockSpec((BLK, 2), lambda i: (i, 0)),
        out_shape=jax.ShapeDtypeStruct((EPAD, 2), jnp.float32),
    )(xf, xf, e_pad, se, te, wa, wb, wc, b1, w2, b2, w3, b3, w4, b4, w5, b5)


def kernel(x, edge_index, e, xbatch, bn_node_g, bn_node_b, bn_edge_g, bn_edge_b,
           nn0_w1, nn0_b1, nn0_w2, nn0_b2, root0, bias0,
           nn1_w1, nn1_b1, nn1_w2, nn1_b2, root1, bias1,
           nn2_w1, nn2_b1, nn2_w2, nn2_b2, root2, bias2,
           ep_w1, ep_b1, ep_w2, ep_b2, ep_w3, ep_b3, ep_w4, ep_b4,
           ep_w5, ep_b5):
    f32 = jnp.float32
    src = edge_index[0]
    dst = edge_index[1]
    pad = EPAD - E
    src2 = jnp.pad(src, (0, pad)).reshape(ROWS, 128)
    dst2g = jnp.pad(dst, (0, pad)).reshape(ROWS, 128)
    dst2s = jnp.pad(dst, (0, pad), constant_values=N).reshape(ROWS, 128)

    x_bn, se_c, te_c = _prep(x, e.T,
                             bn_node_g.reshape(1, -1), bn_node_b.reshape(1, -1),
                             bn_edge_g.reshape(-1, 1), bn_edge_b.reshape(-1, 1))
    se = se_c.reshape(1, -1)
    te = te_c.reshape(1, -1)
    e_pad = jnp.pad(e, ((0, pad), (0, 0)))

    zeros32 = jnp.zeros((NACC, 32), f32)
    layers = [(16, 32, nn0_w1, nn0_b1, nn0_w2, nn0_b2, root0, bias0),
              (32, 32, nn1_w1, nn1_b1, nn1_w2, nn1_b2, root1, bias1),
              (32, 32, nn2_w1, nn2_b1, nn2_w2, nn2_b2, root2, bias2)]
    xcur = x_bn
    for nin, nout, w1, b1, w2, b2, root, bias in layers:
        xg = _sc_gather(xcur, src2, ROWS, nin)
        rm = jnp.repeat(jnp.eye(nin, dtype=f32), nout, axis=1)
        fm = jnp.tile(jnp.eye(nout, dtype=f32), (nin, 1))
        msg = _msg(e_pad, se, te, xg, w1, b1.reshape(1, -1), w2,
                   b2.reshape(1, -1), rm, fm, nin, nout)
        part = _sc_scatter(msg, dst2s, zeros32, nout)
        xcur = _update(part[:, :N, :], xcur, root, bias.reshape(1, -1))

    idxf = jnp.concatenate([src2, dst2g], axis=0)
    xf = _sc_gather(xcur, idxf, 2 * ROWS, 32)
    out = _final(xf, e_pad, se, te,
                 ep_w1[0:32], ep_w1[32:64], ep_w1[64:74], ep_b1.reshape(1, -1),
                 ep_w2, ep_b2.reshape(1, -1), ep_w3, ep_b3.reshape(1, -1),
                 ep_w4, ep_b4.reshape(1, -1), ep_w5, ep_b5.reshape(1, -1))
    return out[:E]


# speedup vs baseline: 3.1166x; 3.1166x over previous
import functools

import jax
import jax.numpy as jnp
from jax import lax
from jax.experimental import pallas as pl
from jax.experimental.pallas import tpu as pltpu
from jax.experimental.pallas import tpu_sc as plsc

N = 10000
E = 160000
EPAD = 163840
ROWS = EPAD // 128
NACC = 10016
NC, NS = 2, 16
NW = NC * NS
RPW = ROWS // NW
LEAK = 0.1
EPS = 1e-5
BLK = 1024


def _leaky(v):
    return jnp.where(v >= 0, v, LEAK * v)


def _dot(a, b):
    return lax.dot_general(a, b, (((1,), (0,)), ((), ())),
                           preferred_element_type=jnp.float32,
                           precision=lax.Precision.DEFAULT)


def _prep(x, et, gx, bx, ge, be):
    def body(x_ref, et_ref, gx_ref, bx_ref, ge_ref, be_ref,
             xo_ref, se_ref, te_ref):
        xv = x_ref[...]
        m = jnp.mean(xv, axis=0, keepdims=True)
        v = jnp.mean((xv - m) ** 2, axis=0, keepdims=True)
        xo_ref[...] = (xv - m) * lax.rsqrt(v + EPS) * gx_ref[...] + bx_ref[...]
        ev = et_ref[...]
        me = jnp.mean(ev, axis=1, keepdims=True)
        ve = jnp.mean((ev - me) ** 2, axis=1, keepdims=True)
        s = lax.rsqrt(ve + EPS) * ge_ref[...]
        se_ref[...] = s
        te_ref[...] = be_ref[...] - me * s

    return pl.pallas_call(
        body,
        out_shape=(jax.ShapeDtypeStruct((N, 16), jnp.float32),
                   jax.ShapeDtypeStruct((10, 1), jnp.float32),
                   jax.ShapeDtypeStruct((10, 1), jnp.float32)),
    )(x, et, gx, bx, ge, be)


def _sc_gather(table, idx2, m_rows, d):
    rpw = m_rows // NW
    nsup = rpw // 8
    mesh = plsc.VectorSubcoreMesh(core_axis_name="c", subcore_axis_name="s")

    @functools.partial(
        pl.kernel,
        out_type=jax.ShapeDtypeStruct((m_rows * 128, d), jnp.float32),
        mesh=mesh,
        compiler_params=pltpu.CompilerParams(use_tc_tiling_on_sc=False),
        scratch_types=[pltpu.VMEM((8, 128), jnp.int32),
                       pltpu.VMEM((128, d), jnp.float32)],
    )
    def k(table_hbm, idx_hbm, out_hbm, idx_v, rows_v):
        c = lax.axis_index("c")
        s = lax.axis_index("s")
        row0 = (s * NC + c) * rpw

        @pl.loop(0, nsup)
        def _sup(jb):
            r0 = row0 + jb * 8
            pltpu.sync_copy(idx_hbm.at[pl.ds(r0, 8)], idx_v)

            @pl.loop(0, 8)
            def _row(r):
                pltpu.sync_copy(table_hbm.at[idx_v.at[r]], rows_v)
                pltpu.sync_copy(rows_v, out_hbm.at[pl.ds((r0 + r) * 128, 128)])

    return k(table, idx2)


def _sc_scatter(msg, dst2, zeros, d):
    mesh = plsc.VectorSubcoreMesh(core_axis_name="c", subcore_axis_name="s")
    rps = NACC // NS

    @functools.partial(
        pl.kernel,
        out_type=jax.ShapeDtypeStruct((NC, NACC, d), jnp.float32),
        mesh=mesh,
        compiler_params=pltpu.CompilerParams(use_tc_tiling_on_sc=False),
        scratch_types=[pltpu.VMEM((8, 128), jnp.int32),
                       pltpu.VMEM((128, d), jnp.float32),
                       pltpu.VMEM_SHARED((NACC, d), jnp.float32)],
    )
    def k(msg_hbm, dst_hbm, z_hbm, out_hbm, idx_v, rows_v, acc_sh):
        c = lax.axis_index("c")
        s = lax.axis_index("s")
        row0 = (s * NC + c) * RPW
        rs = s * rps
        pltpu.sync_copy(z_hbm.at[pl.ds(rs, rps)], acc_sh.at[pl.ds(rs, rps)])
        plsc.subcore_barrier()

        @pl.loop(0, RPW // 8)
        def _sup(jb):
            r0 = row0 + jb * 8
            pltpu.sync_copy(dst_hbm.at[pl.ds(r0, 8)], idx_v)

            @pl.loop(0, 8)
            def _row(r):
                pltpu.sync_copy(msg_hbm.at[pl.ds((r0 + r) * 128, 128)], rows_v)
                pltpu.sync_copy(rows_v, acc_sh.at[idx_v.at[r]], add=True)

        plsc.subcore_barrier()
        pltpu.sync_copy(acc_sh.at[pl.ds(rs, rps)], out_hbm.at[c, pl.ds(rs, rps)])

    return k(msg, dst2, zeros)


def _msg(e_pad, se, te, xg, w1, b1, w2, b2, rm, fm, nin, nout):
    g = EPAD // BLK

    def body(e_ref, se_ref, te_ref, xg_ref, w1_ref, b1_ref, w2_ref, b2_ref,
             r_ref, f_ref, o_ref):
        eb = e_ref[...] * se_ref[...] + te_ref[...]
        h1 = _leaky(_dot(eb, w1_ref[...]) + b1_ref[...])
        w = _leaky(_dot(h1, w2_ref[...]) + b2_ref[...])
        rep = _dot(xg_ref[...], r_ref[...])
        o_ref[...] = _dot(w * rep, f_ref[...])

    k = nin * nout
    return pl.pallas_call(
        body,
        grid=(g,),
        in_specs=[pl.BlockSpec((BLK, 10), lambda i: (i, 0)),
                  pl.BlockSpec((1, 10), lambda i: (0, 0)),
                  pl.BlockSpec((1, 10), lambda i: (0, 0)),
                  pl.BlockSpec((BLK, nin), lambda i: (i, 0)),
                  pl.BlockSpec((10, nin), lambda i: (0, 0)),
                  pl.BlockSpec((1, nin), lambda i: (0, 0)),
                  pl.BlockSpec((nin, k), lambda i: (0, 0)),
                  pl.BlockSpec((1, k), lambda i: (0, 0)),
                  pl.BlockSpec((nin, k), lambda i: (0, 0)),
                  pl.BlockSpec((k, nout), lambda i: (0, 0))],
        out_specs=pl.BlockSpec((BLK, nout), lambda i: (i, 0)),
        out_shape=jax.ShapeDtypeStruct((EPAD, nout), jnp.float32),
        compiler_params=pltpu.CompilerParams(
            dimension_semantics=("parallel",)),
    )(e_pad, se, te, xg, w1, b1, w2, b2, rm, fm)


def _update(part, xin, root, bias):
    nout = root.shape[1]

    def body(p_ref, x_ref, r_ref, b_ref, o_ref):
        o_ref[...] = (p_ref[0] + p_ref[1] + _dot(x_ref[...], r_ref[...])
                      + b_ref[...])

    return pl.pallas_call(
        body,
        out_shape=jax.ShapeDtypeStruct((N, nout), jnp.float32),
    )(part, xin, root, bias)


def _final(xf, e_pad, se, te, wa, wb, wc, b1, w2, b2, w3, b3, w4, b4, w5, b5):
    g = EPAD // BLK

    def body(xs_ref, xd_ref, e_ref, se_ref, te_ref, wa_ref, wb_ref, wc_ref,
             b1_ref, w2_ref, b2_ref, w3_ref, b3_ref, w4_ref, b4_ref,
             w5_ref, b5_ref, o_ref):
        eb = e_ref[...] * se_ref[...] + te_ref[...]
        h = _leaky(_dot(xs_ref[...], wa_ref[...]) + _dot(xd_ref[...], wb_ref[...])
                   + _dot(eb, wc_ref[...]) + b1_ref[...])
        h = _leaky(_dot(h, w2_ref[...]) + b2_ref[...])
        h = _leaky(_dot(h, w3_ref[...]) + b3_ref[...])
        h = _leaky(_dot(h, w4_ref[...]) + b4_ref[...])
        o_ref[...] = _dot(h, w5_ref[...]) + b5_ref[...]

    return pl.pallas_call(
        body,
        grid=(g,),
        in_specs=[pl.BlockSpec((BLK, 32), lambda i: (i, 0)),
                  pl.BlockSpec((BLK, 32), lambda i: (i + g, 0)),
                  pl.BlockSpec((BLK, 10), lambda i: (i, 0)),
                  pl.BlockSpec((1, 10), lambda i: (0, 0)),
                  pl.BlockSpec((1, 10), lambda i: (0, 0)),
                  pl.BlockSpec((32, 64), lambda i: (0, 0)),
                  pl.BlockSpec((32, 64), lambda i: (0, 0)),
                  pl.BlockSpec((10, 64), lambda i: (0, 0)),
                  pl.BlockSpec((1, 64), lambda i: (0, 0)),
                  pl.BlockSpec((64, 32), lambda i: (0, 0)),
                  pl.BlockSpec((1, 32), lambda i: (0, 0)),
                  pl.BlockSpec((32, 16), lambda i: (0, 0)),
                  pl.BlockSpec((1, 16), lambda i: (0, 0)),
                  pl.BlockSpec((16, 8), lambda i: (0, 0)),
                  pl.BlockSpec((1, 8), lambda i: (0, 0)),
                  pl.BlockSpec((8, 2), lambda i: (0, 0)),
                  pl.BlockSpec((1, 2), lambda i: (0, 0))],
        out_specs=pl.BlockSpec((BLK, 2), lambda i: (i, 0)),
        out_shape=jax.ShapeDtypeStruct((EPAD, 2), jnp.float32),
        compiler_params=pltpu.CompilerParams(
            dimension_semantics=("parallel",)),
    )(xf, xf, e_pad, se, te, wa, wb, wc, b1, w2, b2, w3, b3, w4, b4, w5, b5)


def kernel(x, edge_index, e, xbatch, bn_node_g, bn_node_b, bn_edge_g, bn_edge_b,
           nn0_w1, nn0_b1, nn0_w2, nn0_b2, root0, bias0,
           nn1_w1, nn1_b1, nn1_w2, nn1_b2, root1, bias1,
           nn2_w1, nn2_b1, nn2_w2, nn2_b2, root2, bias2,
           ep_w1, ep_b1, ep_w2, ep_b2, ep_w3, ep_b3, ep_w4, ep_b4,
           ep_w5, ep_b5):
    f32 = jnp.float32
    src = edge_index[0]
    dst = edge_index[1]
    pad = EPAD - E
    src2 = jnp.pad(src, (0, pad)).reshape(ROWS, 128)
    dst2g = jnp.pad(dst, (0, pad)).reshape(ROWS, 128)
    dst2s = jnp.pad(dst, (0, pad), constant_values=N).reshape(ROWS, 128)

    x_bn, se_c, te_c = _prep(x, e.T,
                             bn_node_g.reshape(1, -1), bn_node_b.reshape(1, -1),
                             bn_edge_g.reshape(-1, 1), bn_edge_b.reshape(-1, 1))
    se = se_c.reshape(1, -1)
    te = te_c.reshape(1, -1)
    e_pad = jnp.pad(e, ((0, pad), (0, 0)))

    zeros32 = jnp.zeros((NACC, 32), f32)
    layers = [(16, 32, nn0_w1, nn0_b1, nn0_w2, nn0_b2, root0, bias0),
              (32, 32, nn1_w1, nn1_b1, nn1_w2, nn1_b2, root1, bias1),
              (32, 32, nn2_w1, nn2_b1, nn2_w2, nn2_b2, root2, bias2)]
    xcur = x_bn
    for nin, nout, w1, b1, w2, b2, root, bias in layers:
        xg = _sc_gather(xcur, src2, ROWS, nin)
        rm = jnp.repeat(jnp.eye(nin, dtype=f32), nout, axis=1)
        fm = jnp.tile(jnp.eye(nout, dtype=f32), (nin, 1))
        msg = _msg(e_pad, se, te, xg, w1, b1.reshape(1, -1), w2,
                   b2.reshape(1, -1), rm, fm, nin, nout)
        part = _sc_scatter(msg, dst2s, zeros32, nout)
        xcur = _update(part[:, :N, :], xcur, root, bias.reshape(1, -1))

    idxf = jnp.concatenate([src2, dst2g], axis=0)
    xf = _sc_gather(xcur, idxf, 2 * ROWS, 32)
    out = _final(xf, e_pad, se, te,
                 ep_w1[0:32], ep_w1[32:64], ep_w1[64:74], ep_b1.reshape(1, -1),
                 ep_w2, ep_b2.reshape(1, -1), ep_w3, ep_b3.reshape(1, -1),
                 ep_w4, ep_b4.reshape(1, -1), ep_w5, ep_b5.reshape(1, -1))
    return out[:E]


# speedup vs baseline: 3.3462x; 1.0737x over previous
import functools

import jax
import jax.numpy as jnp
from jax import lax
from jax.experimental import pallas as pl
from jax.experimental.pallas import tpu as pltpu
from jax.experimental.pallas import tpu_sc as plsc

N = 10000
E = 160000
EPAD = 163840
ROWS = EPAD // 128
NACC = 10016
NC, NS = 2, 16
NW = NC * NS
RPW = ROWS // NW
LEAK = 0.1
EPS = 1e-5
BLK = 1600


def _leaky(v):
    return jnp.where(v >= 0, v, LEAK * v)


def _dot(a, b):
    return lax.dot_general(a, b, (((1,), (0,)), ((), ())),
                           preferred_element_type=jnp.float32,
                           precision=lax.Precision.DEFAULT)


def _prep(x, et, gx, bx, ge, be):
    def body(x_ref, et_ref, gx_ref, bx_ref, ge_ref, be_ref,
             xo_ref, se_ref, te_ref):
        xv = x_ref[...]
        m = jnp.mean(xv, axis=0, keepdims=True)
        v = jnp.mean((xv - m) ** 2, axis=0, keepdims=True)
        xo_ref[...] = (xv - m) * lax.rsqrt(v + EPS) * gx_ref[...] + bx_ref[...]
        ev = et_ref[...]
        me = jnp.mean(ev, axis=1, keepdims=True)
        ve = jnp.mean((ev - me) ** 2, axis=1, keepdims=True)
        s = lax.rsqrt(ve + EPS) * ge_ref[...]
        se_ref[...] = s
        te_ref[...] = be_ref[...] - me * s

    return pl.pallas_call(
        body,
        out_shape=(jax.ShapeDtypeStruct((N, 16), jnp.float32),
                   jax.ShapeDtypeStruct((10, 1), jnp.float32),
                   jax.ShapeDtypeStruct((10, 1), jnp.float32)),
    )(x, et, gx, bx, ge, be)


def _sc_gather(table, idx2s, d):
    n_out = len(idx2s)
    mesh = plsc.VectorSubcoreMesh(core_axis_name="c", subcore_axis_name="s")

    @functools.partial(
        pl.kernel,
        out_type=tuple(jax.ShapeDtypeStruct((EPAD, d), jnp.float32)
                       for _ in range(n_out)),
        mesh=mesh,
        compiler_params=pltpu.CompilerParams(use_tc_tiling_on_sc=False),
        scratch_types=[pltpu.VMEM((8, 128), jnp.int32),
                       pltpu.VMEM((128, d), jnp.float32)],
    )
    def k(table_hbm, *refs):
        idx_hbms = refs[:n_out]
        out_hbms = refs[n_out:2 * n_out]
        idx_v, rows_v = refs[2 * n_out:]
        c = lax.axis_index("c")
        s = lax.axis_index("s")
        row0 = (s * NC + c) * RPW
        for idx_hbm, out_hbm in zip(idx_hbms, out_hbms):
            @pl.loop(0, RPW // 8)
            def _sup(jb):
                r0 = row0 + jb * 8
                pltpu.sync_copy(idx_hbm.at[pl.ds(r0, 8)], idx_v)

                @pl.loop(0, 8)
                def _row(r):
                    pltpu.sync_copy(table_hbm.at[idx_v.at[r]], rows_v)
                    pltpu.sync_copy(rows_v,
                                    out_hbm.at[pl.ds((r0 + r) * 128, 128)])

    outs = k(table, *idx2s)
    return list(outs) if isinstance(outs, (tuple, list)) else [outs]


def _sc_scatter(msgp, dst2, zeros, d):
    mesh = plsc.VectorSubcoreMesh(core_axis_name="c", subcore_axis_name="s")
    rps = NACC // NS

    @functools.partial(
        pl.kernel,
        out_type=jax.ShapeDtypeStruct((NC, NACC, d), jnp.float32),
        mesh=mesh,
        compiler_params=pltpu.CompilerParams(use_tc_tiling_on_sc=False),
        scratch_types=[pltpu.VMEM((8, 128), jnp.int32),
                       pltpu.VMEM((128, d), jnp.float32),
                       pltpu.VMEM_SHARED((NACC, d), jnp.float32)],
    )
    def k(msg_hbm, dst_hbm, z_hbm, out_hbm, idx_v, rows_v, acc_sh):
        c = lax.axis_index("c")
        s = lax.axis_index("s")
        row0 = (s * NC + c) * RPW
        rs = s * rps
        pltpu.sync_copy(z_hbm.at[pl.ds(rs, rps)], acc_sh.at[pl.ds(rs, rps)])
        plsc.subcore_barrier()

        @pl.loop(0, RPW // 8)
        def _sup(jb):
            r0 = row0 + jb * 8
            pltpu.sync_copy(dst_hbm.at[pl.ds(r0, 8)], idx_v)

            @pl.loop(0, 8)
            def _row(r):
                pltpu.sync_copy(msg_hbm.at[pl.ds((r0 + r) * 128, 128)], rows_v)
                pltpu.sync_copy(rows_v, acc_sh.at[idx_v.at[r]], add=True)

        plsc.subcore_barrier()
        pltpu.sync_copy(acc_sh.at[pl.ds(rs, rps)], out_hbm.at[c, pl.ds(rs, rps)])

    return k(msgp, dst2, zeros)


def _msg(e, se, te, xg1, w1, b1, w2, b2, rm, fm, nin, nout):
    g = E // BLK

    def body(e_ref, se_ref, te_ref, xg_ref, w1_ref, b1_ref, w2_ref, b2_ref,
             r_ref, f_ref, o_ref):
        eb = e_ref[...] * se_ref[...] + te_ref[...]
        h1 = _leaky(_dot(eb, w1_ref[...]) + b1_ref[...])
        w = _leaky(_dot(h1, w2_ref[...]) + b2_ref[...])
        rep = _dot(xg_ref[...], r_ref[...])
        o_ref[...] = _dot(w * rep, f_ref[...])

    k = nin * nout
    return pl.pallas_call(
        body,
        grid=(g,),
        in_specs=[pl.BlockSpec((BLK, 10), lambda i: (i, 0)),
                  pl.BlockSpec((1, 10), lambda i: (0, 0)),
                  pl.BlockSpec((1, 10), lambda i: (0, 0)),
                  pl.BlockSpec((BLK, nin), lambda i: (i, 0)),
                  pl.BlockSpec((10, nin), lambda i: (0, 0)),
                  pl.BlockSpec((1, nin), lambda i: (0, 0)),
                  pl.BlockSpec((nin, k), lambda i: (0, 0)),
                  pl.BlockSpec((1, k), lambda i: (0, 0)),
                  pl.BlockSpec((nin, k), lambda i: (0, 0)),
                  pl.BlockSpec((k, nout), lambda i: (0, 0))],
        out_specs=pl.BlockSpec((BLK, nout), lambda i: (i, 0)),
        out_shape=jax.ShapeDtypeStruct((EPAD, nout), jnp.float32),
        compiler_params=pltpu.CompilerParams(
            dimension_semantics=("parallel",)),
    )(e, se, te, xg1, w1, b1, w2, b2, rm, fm)


def _update(part, xin, root, bias):
    nout = root.shape[1]

    def body(p_ref, x_ref, r_ref, b_ref, o_ref):
        o_ref[...] = (p_ref[0] + p_ref[1] + _dot(x_ref[...], r_ref[...])
                      + b_ref[...])

    return pl.pallas_call(
        body,
        out_shape=jax.ShapeDtypeStruct((N, nout), jnp.float32),
    )(part, xin, root, bias)


def _final(xsp, xdp, e, se, te, wa, wb, wc, b1, w2, b2, w3, b3, w4, b4, w5, b5):
    g = E // BLK

    def body(xs_ref, xd_ref, e_ref, se_ref, te_ref, wa_ref, wb_ref, wc_ref,
             b1_ref, w2_ref, b2_ref, w3_ref, b3_ref, w4_ref, b4_ref,
             w5_ref, b5_ref, o_ref):
        eb = e_ref[...] * se_ref[...] + te_ref[...]
        xs = xs_ref[...]
        xd = xd_ref[...]
        h = _leaky(_dot(xs, wa_ref[...]) + _dot(xd, wb_ref[...])
                   + _dot(eb, wc_ref[...]) + b1_ref[...])
        h = _leaky(_dot(h, w2_ref[...]) + b2_ref[...])
        h = _leaky(_dot(h, w3_ref[...]) + b3_ref[...])
        h = _leaky(_dot(h, w4_ref[...]) + b4_ref[...])
        o_ref[...] = _dot(h, w5_ref[...]) + b5_ref[...]

    return pl.pallas_call(
        body,
        grid=(g,),
        in_specs=[pl.BlockSpec((BLK, 32), lambda i: (i, 0)),
                  pl.BlockSpec((BLK, 32), lambda i: (i, 0)),
                  pl.BlockSpec((BLK, 10), lambda i: (i, 0)),
                  pl.BlockSpec((1, 10), lambda i: (0, 0)),
                  pl.BlockSpec((1, 10), lambda i: (0, 0)),
                  pl.BlockSpec((32, 64), lambda i: (0, 0)),
                  pl.BlockSpec((32, 64), lambda i: (0, 0)),
                  pl.BlockSpec((10, 64), lambda i: (0, 0)),
                  pl.BlockSpec((1, 64), lambda i: (0, 0)),
                  pl.BlockSpec((64, 32), lambda i: (0, 0)),
                  pl.BlockSpec((1, 32), lambda i: (0, 0)),
                  pl.BlockSpec((32, 16), lambda i: (0, 0)),
                  pl.BlockSpec((1, 16), lambda i: (0, 0)),
                  pl.BlockSpec((16, 8), lambda i: (0, 0)),
                  pl.BlockSpec((1, 8), lambda i: (0, 0)),
                  pl.BlockSpec((8, 2), lambda i: (0, 0)),
                  pl.BlockSpec((1, 2), lambda i: (0, 0))],
        out_specs=pl.BlockSpec((BLK, 2), lambda i: (i, 0)),
        out_shape=jax.ShapeDtypeStruct((E, 2), jnp.float32),
        compiler_params=pltpu.CompilerParams(
            dimension_semantics=("parallel",)),
    )(xsp, xdp, e, se, te, wa, wb, wc, b1, w2, b2, w3, b3, w4, b4, w5, b5)


def kernel(x, edge_index, e, xbatch, bn_node_g, bn_node_b, bn_edge_g, bn_edge_b,
           nn0_w1, nn0_b1, nn0_w2, nn0_b2, root0, bias0,
           nn1_w1, nn1_b1, nn1_w2, nn1_b2, root1, bias1,
           nn2_w1, nn2_b1, nn2_w2, nn2_b2, root2, bias2,
           ep_w1, ep_b1, ep_w2, ep_b2, ep_w3, ep_b3, ep_w4, ep_b4,
           ep_w5, ep_b5):
    f32 = jnp.float32
    src = edge_index[0]
    dst = edge_index[1]
    pad = EPAD - E
    src2 = jnp.pad(src, (0, pad)).reshape(ROWS, 128)
    dst2g = jnp.pad(dst, (0, pad)).reshape(ROWS, 128)
    dst2s = jnp.pad(dst, (0, pad), constant_values=N).reshape(ROWS, 128)

    x_bn, se_c, te_c = _prep(x, e.T,
                             bn_node_g.reshape(1, -1), bn_node_b.reshape(1, -1),
                             bn_edge_g.reshape(-1, 1), bn_edge_b.reshape(-1, 1))
    se = se_c.reshape(1, -1)
    te = te_c.reshape(1, -1)

    zeros32 = jnp.zeros((NACC, 32), f32)
    layers = [(16, 32, nn0_w1, nn0_b1, nn0_w2, nn0_b2, root0, bias0),
              (32, 32, nn1_w1, nn1_b1, nn1_w2, nn1_b2, root1, bias1),
              (32, 32, nn2_w1, nn2_b1, nn2_w2, nn2_b2, root2, bias2)]
    xcur = x_bn
    for nin, nout, w1, b1, w2, b2, root, bias in layers:
        (xg,) = _sc_gather(xcur, [src2], nin)
        rm = jnp.repeat(jnp.eye(nin, dtype=f32), nout, axis=1)
        fm = jnp.tile(jnp.eye(nout, dtype=f32), (nin, 1))
        msg1 = _msg(e, se, te, xg, w1, b1.reshape(1, -1), w2,
                    b2.reshape(1, -1), rm, fm, nin, nout)
        part = _sc_scatter(msg1, dst2s, zeros32, nout)
        xcur = _update(part[:, :N, :], xcur, root, bias.reshape(1, -1))

    xs, xd = _sc_gather(xcur, [src2, dst2g], 32)
    out = _final(xs, xd, e, se, te,
                 ep_w1[0:32], ep_w1[32:64], ep_w1[64:74], ep_b1.reshape(1, -1),
                 ep_w2, ep_b2.reshape(1, -1), ep_w3, ep_b3.reshape(1, -1),
                 ep_w4, ep_b4.reshape(1, -1), ep_w5, ep_b5.reshape(1, -1))
    return out


# speedup vs baseline: 3.4750x; 1.0385x over previous
import functools

import jax
import jax.numpy as jnp
from jax import lax
from jax.experimental import pallas as pl
from jax.experimental.pallas import tpu as pltpu
from jax.experimental.pallas import tpu_sc as plsc

N = 10000
E = 160000
EPAD = 163840
ROWS = EPAD // 128
NQ = 4
QROWS = ROWS // NQ
QE = QROWS * 128
NACC = 10016
NC, NS = 2, 16
NW = NC * NS
RPQ = QROWS // NW
LEAK = 0.1
EPS = 1e-5
BLK = 1280


def _leaky(v):
    return jnp.where(v >= 0, v, LEAK * v)


def _dot(a, b):
    return lax.dot_general(a, b, (((1,), (0,)), ((), ())),
                           preferred_element_type=jnp.float32,
                           precision=lax.Precision.DEFAULT)


def _nblk(q):
    lo = q * QE
    hi = min((q + 1) * QE, E)
    assert (hi - lo) % BLK == 0
    return (hi - lo) // BLK


def _prep(x, et, gx, bx, ge, be):
    def body(x_ref, et_ref, gx_ref, bx_ref, ge_ref, be_ref,
             xo_ref, se_ref, te_ref):
        xv = x_ref[...]
        m = jnp.mean(xv, axis=0, keepdims=True)
        v = jnp.mean((xv - m) ** 2, axis=0, keepdims=True)
        xo_ref[...] = (xv - m) * lax.rsqrt(v + EPS) * gx_ref[...] + bx_ref[...]
        ev = et_ref[...]
        me = jnp.mean(ev, axis=1, keepdims=True)
        ve = jnp.mean((ev - me) ** 2, axis=1, keepdims=True)
        s = lax.rsqrt(ve + EPS) * ge_ref[...]
        se_ref[...] = s
        te_ref[...] = be_ref[...] - me * s

    return pl.pallas_call(
        body,
        out_shape=(jax.ShapeDtypeStruct((N, 16), jnp.float32),
                   jax.ShapeDtypeStruct((10, 1), jnp.float32),
                   jax.ShapeDtypeStruct((10, 1), jnp.float32)),
    )(x, et, gx, bx, ge, be)


def _sc_gather(table, idx2s, d):
    n_out = len(idx2s)
    mesh = plsc.VectorSubcoreMesh(core_axis_name="c", subcore_axis_name="s")

    @functools.partial(
        pl.kernel,
        out_type=tuple(jax.ShapeDtypeStruct((QE, d), jnp.float32)
                       for _ in range(n_out)),
        mesh=mesh,
        compiler_params=pltpu.CompilerParams(use_tc_tiling_on_sc=False),
        scratch_types=[pltpu.VMEM((RPQ, 128), jnp.int32),
                       pltpu.VMEM((128, d), jnp.float32)],
    )
    def k(table_hbm, *refs):
        idx_hbms = refs[:n_out]
        out_hbms = refs[n_out:2 * n_out]
        idx_v, rows_v = refs[2 * n_out:]
        c = lax.axis_index("c")
        s = lax.axis_index("s")
        row0 = (s * NC + c) * RPQ
        for idx_hbm, out_hbm in zip(idx_hbms, out_hbms):
            pltpu.sync_copy(idx_hbm.at[pl.ds(row0, RPQ)], idx_v)

            @pl.loop(0, RPQ)
            def _row(r):
                pltpu.sync_copy(table_hbm.at[idx_v.at[r]], rows_v)
                pltpu.sync_copy(rows_v,
                                out_hbm.at[pl.ds((row0 + r) * 128, 128)])

    outs = k(table, *idx2s)
    return list(outs) if isinstance(outs, (tuple, list)) else [outs]


def _sc_scatter(msgq, dst2q, zeros, d):
    mesh = plsc.VectorSubcoreMesh(core_axis_name="c", subcore_axis_name="s")
    rps = NACC // NS

    @functools.partial(
        pl.kernel,
        out_type=jax.ShapeDtypeStruct((NC, NACC, d), jnp.float32),
        mesh=mesh,
        compiler_params=pltpu.CompilerParams(use_tc_tiling_on_sc=False),
        scratch_types=[pltpu.VMEM((RPQ, 128), jnp.int32),
                       pltpu.VMEM((128, d), jnp.float32),
                       pltpu.VMEM_SHARED((NACC, d), jnp.float32)],
    )
    def k(msg_hbm, dst_hbm, z_hbm, out_hbm, idx_v, rows_v, acc_sh):
        c = lax.axis_index("c")
        s = lax.axis_index("s")
        row0 = (s * NC + c) * RPQ
        rs = s * rps
        pltpu.sync_copy(z_hbm.at[pl.ds(rs, rps)], acc_sh.at[pl.ds(rs, rps)])
        pltpu.sync_copy(dst_hbm.at[pl.ds(row0, RPQ)], idx_v)
        plsc.subcore_barrier()

        @pl.loop(0, RPQ)
        def _row(r):
            pltpu.sync_copy(msg_hbm.at[pl.ds((row0 + r) * 128, 128)], rows_v)
            pltpu.sync_copy(rows_v, acc_sh.at[idx_v.at[r]], add=True)

        plsc.subcore_barrier()
        pltpu.sync_copy(acc_sh.at[pl.ds(rs, rps)], out_hbm.at[c, pl.ds(rs, rps)])

    return k(msgq, dst2q, zeros)


def _msg(e, se, te, xgq, w1, b1, w2, b2, rm, fm, nin, nout, q):
    g = _nblk(q)
    eoff = q * QE // BLK

    def body(e_ref, se_ref, te_ref, xg_ref, w1_ref, b1_ref, w2_ref, b2_ref,
             r_ref, f_ref, o_ref):
        eb = e_ref[...] * se_ref[...] + te_ref[...]
        h1 = _leaky(_dot(eb, w1_ref[...]) + b1_ref[...])
        w = _leaky(_dot(h1, w2_ref[...]) + b2_ref[...])
        rep = _dot(xg_ref[...], r_ref[...])
        o_ref[...] = _dot(w * rep, f_ref[...])

    k = nin * nout
    return pl.pallas_call(
        body,
        grid=(g,),
        in_specs=[pl.BlockSpec((BLK, 10), lambda i: (i + eoff, 0)),
                  pl.BlockSpec((1, 10), lambda i: (0, 0)),
                  pl.BlockSpec((1, 10), lambda i: (0, 0)),
                  pl.BlockSpec((BLK, nin), lambda i: (i, 0)),
                  pl.BlockSpec((10, nin), lambda i: (0, 0)),
                  pl.BlockSpec((1, nin), lambda i: (0, 0)),
                  pl.BlockSpec((nin, k), lambda i: (0, 0)),
                  pl.BlockSpec((1, k), lambda i: (0, 0)),
                  pl.BlockSpec((nin, k), lambda i: (0, 0)),
                  pl.BlockSpec((k, nout), lambda i: (0, 0))],
        out_specs=pl.BlockSpec((BLK, nout), lambda i: (i, 0)),
        out_shape=jax.ShapeDtypeStruct((QE, nout), jnp.float32),
        compiler_params=pltpu.CompilerParams(
            dimension_semantics=("parallel",)),
    )(e, se, te, xgq, w1, b1, w2, b2, rm, fm)


def _update(parts, xin, root, bias):
    nout = root.shape[1]
    nparts = len(parts)

    def body(*refs):
        p_refs = refs[:nparts]
        x_ref, r_ref, b_ref, o_ref = refs[nparts:]
        acc = _dot(x_ref[...], r_ref[...]) + b_ref[...]
        for p in p_refs:
            acc = acc + p[0] + p[1]
        o_ref[...] = acc

    return pl.pallas_call(
        body,
        out_shape=jax.ShapeDtypeStruct((N, nout), jnp.float32),
    )(*parts, xin, root, bias)


def _final(xsq, xdq, e, se, te, wa, wb, wc, b1, w2, b2, w3, b3, w4, b4,
           w5, b5, q):
    g = _nblk(q)
    eoff = q * QE // BLK

    def body(xs_ref, xd_ref, e_ref, se_ref, te_ref, wa_ref, wb_ref, wc_ref,
             b1_ref, w2_ref, b2_ref, w3_ref, b3_ref, w4_ref, b4_ref,
             w5_ref, b5_ref, o_ref):
        eb = e_ref[...] * se_ref[...] + te_ref[...]
        h = _leaky(_dot(xs_ref[...], wa_ref[...])
                   + _dot(xd_ref[...], wb_ref[...])
                   + _dot(eb, wc_ref[...]) + b1_ref[...])
        h = _leaky(_dot(h, w2_ref[...]) + b2_ref[...])
        h = _leaky(_dot(h, w3_ref[...]) + b3_ref[...])
        h = _leaky(_dot(h, w4_ref[...]) + b4_ref[...])
        o_ref[...] = _dot(h, w5_ref[...]) + b5_ref[...]

    return pl.pallas_call(
        body,
        grid=(g,),
        in_specs=[pl.BlockSpec((BLK, 32), lambda i: (i, 0)),
                  pl.BlockSpec((BLK, 32), lambda i: (i, 0)),
                  pl.BlockSpec((BLK, 10), lambda i: (i + eoff, 0)),
                  pl.BlockSpec((1, 10), lambda i: (0, 0)),
                  pl.BlockSpec((1, 10), lambda i: (0, 0)),
                  pl.BlockSpec((32, 64), lambda i: (0, 0)),
                  pl.BlockSpec((32, 64), lambda i: (0, 0)),
                  pl.BlockSpec((10, 64), lambda i: (0, 0)),
                  pl.BlockSpec((1, 64), lambda i: (0, 0)),
                  pl.BlockSpec((64, 32), lambda i: (0, 0)),
                  pl.BlockSpec((1, 32), lambda i: (0, 0)),
                  pl.BlockSpec((32, 16), lambda i: (0, 0)),
                  pl.BlockSpec((1, 16), lambda i: (0, 0)),
                  pl.BlockSpec((16, 8), lambda i: (0, 0)),
                  pl.BlockSpec((1, 8), lambda i: (0, 0)),
                  pl.BlockSpec((8, 2), lambda i: (0, 0)),
                  pl.BlockSpec((1, 2), lambda i: (0, 0))],
        out_specs=pl.BlockSpec((BLK, 2), lambda i: (i, 0)),
        out_shape=jax.ShapeDtypeStruct((g * BLK, 2), jnp.float32),
        compiler_params=pltpu.CompilerParams(
            dimension_semantics=("parallel",)),
    )(xsq, xdq, e, se, te, wa, wb, wc, b1, w2, b2, w3, b3, w4, b4, w5, b5)


def kernel(x, edge_index, e, xbatch, bn_node_g, bn_node_b, bn_edge_g, bn_edge_b,
           nn0_w1, nn0_b1, nn0_w2, nn0_b2, root0, bias0,
           nn1_w1, nn1_b1, nn1_w2, nn1_b2, root1, bias1,
           nn2_w1, nn2_b1, nn2_w2, nn2_b2, root2, bias2,
           ep_w1, ep_b1, ep_w2, ep_b2, ep_w3, ep_b3, ep_w4, ep_b4,
           ep_w5, ep_b5):
    f32 = jnp.float32
    src = edge_index[0]
    dst = edge_index[1]
    pad = EPAD - E
    src2 = jnp.pad(src, (0, pad)).reshape(ROWS, 128)
    dst2g = jnp.pad(dst, (0, pad)).reshape(ROWS, 128)
    dst2s = jnp.pad(dst, (0, pad), constant_values=N).reshape(ROWS, 128)
    srcq = [src2[q * QROWS:(q + 1) * QROWS] for q in range(NQ)]
    dstgq = [dst2g[q * QROWS:(q + 1) * QROWS] for q in range(NQ)]
    dstsq = [dst2s[q * QROWS:(q + 1) * QROWS] for q in range(NQ)]

    x_bn, se_c, te_c = _prep(x, e.T,
                             bn_node_g.reshape(1, -1), bn_node_b.reshape(1, -1),
                             bn_edge_g.reshape(-1, 1), bn_edge_b.reshape(-1, 1))
    se = se_c.reshape(1, -1)
    te = te_c.reshape(1, -1)

    zeros32 = jnp.zeros((NACC, 32), f32)
    layers = [(16, 32, nn0_w1, nn0_b1, nn0_w2, nn0_b2, root0, bias0),
              (32, 32, nn1_w1, nn1_b1, nn1_w2, nn1_b2, root1, bias1),
              (32, 32, nn2_w1, nn2_b1, nn2_w2, nn2_b2, root2, bias2)]
    xcur = x_bn
    for nin, nout, w1, b1, w2, b2, root, bias in layers:
        rm = jnp.repeat(jnp.eye(nin, dtype=f32), nout, axis=1)
        fm = jnp.tile(jnp.eye(nout, dtype=f32), (nin, 1))
        b1r = b1.reshape(1, -1)
        b2r = b2.reshape(1, -1)
        parts = []
        for q in range(NQ):
            (xg,) = _sc_gather(xcur, [srcq[q]], nin)
            msg = _msg(e, se, te, xg, w1, b1r, w2, b2r, rm, fm, nin, nout, q)
            parts.append(_sc_scatter(msg, dstsq[q], zeros32, nout))
        xcur = _update([p[:, :N, :] for p in parts], xcur, root,
                       bias.reshape(1, -1))

    outs = []
    for q in range(NQ):
        xs, xd = _sc_gather(xcur, [srcq[q], dstgq[q]], 32)
        outs.append(_final(xs, xd, e, se, te,
                           ep_w1[0:32], ep_w1[32:64], ep_w1[64:74],
                           ep_b1.reshape(1, -1),
                           ep_w2, ep_b2.reshape(1, -1), ep_w3,
                           ep_b3.reshape(1, -1), ep_w4, ep_b4.reshape(1, -1),
                           ep_w5, ep_b5.reshape(1, -1), q))
    return jnp.concatenate(outs, axis=0)


# speedup vs baseline: 3.5422x; 1.0193x over previous
import functools

import jax
import jax.numpy as jnp
from jax import lax
from jax.experimental import pallas as pl
from jax.experimental.pallas import tpu as pltpu
from jax.experimental.pallas import tpu_sc as plsc

N = 10000
E = 160000
EPAD = 163840
ROWS = EPAD // 128
NQ = 4
QROWS = ROWS // NQ
QE = QROWS * 128
NACC = 10016
NC, NS = 2, 16
NW = NC * NS
RPQ = QROWS // NW
LEAK = 0.1
EPS = 1e-5
BLK = 1280


def _leaky(v):
    return jnp.where(v >= 0, v, LEAK * v)


def _dot(a, b):
    return lax.dot_general(a, b, (((1,), (0,)), ((), ())),
                           preferred_element_type=jnp.float32,
                           precision=lax.Precision.DEFAULT)


def _nblk(q):
    lo = q * QE
    hi = min((q + 1) * QE, E)
    assert (hi - lo) % BLK == 0
    return (hi - lo) // BLK


def _prep(x, et, gx, bx, ge, be):
    def body(x_ref, et_ref, gx_ref, bx_ref, ge_ref, be_ref,
             xo_ref, se_ref, te_ref):
        xv = x_ref[...]
        m = jnp.mean(xv, axis=0, keepdims=True)
        v = jnp.mean((xv - m) ** 2, axis=0, keepdims=True)
        xo_ref[...] = (xv - m) * lax.rsqrt(v + EPS) * gx_ref[...] + bx_ref[...]
        ev = et_ref[...]
        me = jnp.mean(ev, axis=1, keepdims=True)
        ve = jnp.mean((ev - me) ** 2, axis=1, keepdims=True)
        s = lax.rsqrt(ve + EPS) * ge_ref[...]
        se_ref[...] = s
        te_ref[...] = be_ref[...] - me * s

    return pl.pallas_call(
        body,
        out_shape=(jax.ShapeDtypeStruct((N, 16), jnp.float32),
                   jax.ShapeDtypeStruct((10, 1), jnp.float32),
                   jax.ShapeDtypeStruct((10, 1), jnp.float32)),
    )(x, et, gx, bx, ge, be)


def _sc_gather(table, idx2s, d):
    n_out = len(idx2s)
    mesh = plsc.VectorSubcoreMesh(core_axis_name="c", subcore_axis_name="s")

    @functools.partial(
        pl.kernel,
        out_type=tuple(jax.ShapeDtypeStruct((QE, d), jnp.float32)
                       for _ in range(n_out)),
        mesh=mesh,
        compiler_params=pltpu.CompilerParams(use_tc_tiling_on_sc=False),
        scratch_types=[pltpu.VMEM((RPQ, 128), jnp.int32),
                       pltpu.VMEM((128, d), jnp.float32)],
    )
    def k(table_hbm, *refs):
        idx_hbms = refs[:n_out]
        out_hbms = refs[n_out:2 * n_out]
        idx_v, rows_v = refs[2 * n_out:]
        c = lax.axis_index("c")
        s = lax.axis_index("s")
        row0 = (s * NC + c) * RPQ
        for idx_hbm, out_hbm in zip(idx_hbms, out_hbms):
            pltpu.sync_copy(idx_hbm.at[pl.ds(row0, RPQ)], idx_v)

            @pl.loop(0, RPQ)
            def _row(r):
                pltpu.sync_copy(table_hbm.at[idx_v.at[r]], rows_v)
                pltpu.sync_copy(rows_v,
                                out_hbm.at[pl.ds((row0 + r) * 128, 128)])

    outs = k(table, *idx2s)
    return list(outs) if isinstance(outs, (tuple, list)) else [outs]


def _sc_scatter(msgq, dst2q, zeros, d):
    mesh = plsc.VectorSubcoreMesh(core_axis_name="c", subcore_axis_name="s")
    rps = NACC // NS

    @functools.partial(
        pl.kernel,
        out_type=jax.ShapeDtypeStruct((NC, NACC, d), jnp.float32),
        mesh=mesh,
        compiler_params=pltpu.CompilerParams(use_tc_tiling_on_sc=False),
        scratch_types=[pltpu.VMEM((RPQ, 128), jnp.int32),
                       pltpu.VMEM((128, d), jnp.float32),
                       pltpu.VMEM_SHARED((NACC, d), jnp.float32)],
    )
    def k(msg_hbm, dst_hbm, z_hbm, out_hbm, idx_v, rows_v, acc_sh):
        c = lax.axis_index("c")
        s = lax.axis_index("s")
        row0 = (s * NC + c) * RPQ
        rs = s * rps
        pltpu.sync_copy(z_hbm.at[pl.ds(rs, rps)], acc_sh.at[pl.ds(rs, rps)])
        pltpu.sync_copy(dst_hbm.at[pl.ds(row0, RPQ)], idx_v)
        plsc.subcore_barrier()

        @pl.loop(0, RPQ)
        def _row(r):
            pltpu.sync_copy(msg_hbm.at[pl.ds((row0 + r) * 128, 128)], rows_v)
            pltpu.sync_copy(rows_v, acc_sh.at[idx_v.at[r]], add=True)

        plsc.subcore_barrier()
        pltpu.sync_copy(acc_sh.at[pl.ds(rs, rps)], out_hbm.at[c, pl.ds(rs, rps)])

    return k(msgq, dst2q, zeros)


def _msg(e, se, te, xgq, w1, b1, w2, b2, rm, fm, nin, nout, q):
    g = _nblk(q)
    eoff = q * QE // BLK

    def body(e_ref, se_ref, te_ref, xg_ref, w1_ref, b1_ref, w2_ref, b2_ref,
             r_ref, f_ref, o_ref):
        eb = e_ref[...] * se_ref[...] + te_ref[...]
        h1 = _leaky(_dot(eb, w1_ref[...]) + b1_ref[...])
        w = _leaky(_dot(h1, w2_ref[...]) + b2_ref[...])
        rep = _dot(xg_ref[...], r_ref[...])
        o_ref[...] = _dot(w * rep, f_ref[...])

    k = nin * nout
    return pl.pallas_call(
        body,
        grid=(g,),
        in_specs=[pl.BlockSpec((BLK, 10), lambda i: (i + eoff, 0)),
                  pl.BlockSpec((1, 10), lambda i: (0, 0)),
                  pl.BlockSpec((1, 10), lambda i: (0, 0)),
                  pl.BlockSpec((BLK, nin), lambda i: (i, 0)),
                  pl.BlockSpec((10, nin), lambda i: (0, 0)),
                  pl.BlockSpec((1, nin), lambda i: (0, 0)),
                  pl.BlockSpec((nin, k), lambda i: (0, 0)),
                  pl.BlockSpec((1, k), lambda i: (0, 0)),
                  pl.BlockSpec((nin, k), lambda i: (0, 0)),
                  pl.BlockSpec((k, nout), lambda i: (0, 0))],
        out_specs=pl.BlockSpec((BLK, nout), lambda i: (i, 0)),
        out_shape=jax.ShapeDtypeStruct((QE, nout), jnp.float32),
        compiler_params=pltpu.CompilerParams(
            dimension_semantics=("parallel",)),
    )(e, se, te, xgq, w1, b1, w2, b2, rm, fm)


def _update(parts, xin, root, bias):
    nout = root.shape[1]
    nparts = len(parts)

    def body(*refs):
        p_refs = refs[:nparts]
        x_ref, r_ref, b_ref, o_ref = refs[nparts:]
        acc = _dot(x_ref[...], r_ref[...]) + b_ref[...]
        for p in p_refs:
            acc = acc + p[0] + p[1]
        o_ref[...] = acc

    pspec = pl.BlockSpec((NC, N, nout), lambda i: (0, 0, 0))
    return pl.pallas_call(
        body,
        grid=(1,),
        in_specs=[pspec] * nparts + [
            pl.BlockSpec(xin.shape, lambda i: (0, 0)),
            pl.BlockSpec(root.shape, lambda i: (0, 0)),
            pl.BlockSpec(bias.shape, lambda i: (0, 0))],
        out_specs=pl.BlockSpec((N, nout), lambda i: (0, 0)),
        out_shape=jax.ShapeDtypeStruct((N, nout), jnp.float32),
    )(*parts, xin, root, bias)


def _final(xsq, xdq, e, se, te, wa, wb, wc, b1, w2, b2, w3, b3, w4, b4,
           w5, b5, q):
    g = _nblk(q)
    eoff = q * QE // BLK

    def body(xs_ref, xd_ref, e_ref, se_ref, te_ref, wa_ref, wb_ref, wc_ref,
             b1_ref, w2_ref, b2_ref, w3_ref, b3_ref, w4_ref, b4_ref,
             w5_ref, b5_ref, o_ref):
        eb = e_ref[...] * se_ref[...] + te_ref[...]
        h = _leaky(_dot(xs_ref[...], wa_ref[...])
                   + _dot(xd_ref[...], wb_ref[...])
                   + _dot(eb, wc_ref[...]) + b1_ref[...])
        h = _leaky(_dot(h, w2_ref[...]) + b2_ref[...])
        h = _leaky(_dot(h, w3_ref[...]) + b3_ref[...])
        h = _leaky(_dot(h, w4_ref[...]) + b4_ref[...])
        o_ref[...] = _dot(h, w5_ref[...]) + b5_ref[...]

    return pl.pallas_call(
        body,
        grid=(g,),
        in_specs=[pl.BlockSpec((BLK, 32), lambda i: (i, 0)),
                  pl.BlockSpec((BLK, 32), lambda i: (i, 0)),
                  pl.BlockSpec((BLK, 10), lambda i: (i + eoff, 0)),
                  pl.BlockSpec((1, 10), lambda i: (0, 0)),
                  pl.BlockSpec((1, 10), lambda i: (0, 0)),
                  pl.BlockSpec((32, 64), lambda i: (0, 0)),
                  pl.BlockSpec((32, 64), lambda i: (0, 0)),
                  pl.BlockSpec((10, 64), lambda i: (0, 0)),
                  pl.BlockSpec((1, 64), lambda i: (0, 0)),
                  pl.BlockSpec((64, 32), lambda i: (0, 0)),
                  pl.BlockSpec((1, 32), lambda i: (0, 0)),
                  pl.BlockSpec((32, 16), lambda i: (0, 0)),
                  pl.BlockSpec((1, 16), lambda i: (0, 0)),
                  pl.BlockSpec((16, 8), lambda i: (0, 0)),
                  pl.BlockSpec((1, 8), lambda i: (0, 0)),
                  pl.BlockSpec((8, 2), lambda i: (0, 0)),
                  pl.BlockSpec((1, 2), lambda i: (0, 0))],
        out_specs=pl.BlockSpec((BLK, 2), lambda i: (i, 0)),
        out_shape=jax.ShapeDtypeStruct((g * BLK, 2), jnp.float32),
        compiler_params=pltpu.CompilerParams(
            dimension_semantics=("parallel",)),
    )(xsq, xdq, e, se, te, wa, wb, wc, b1, w2, b2, w3, b3, w4, b4, w5, b5)


def kernel(x, edge_index, e, xbatch, bn_node_g, bn_node_b, bn_edge_g, bn_edge_b,
           nn0_w1, nn0_b1, nn0_w2, nn0_b2, root0, bias0,
           nn1_w1, nn1_b1, nn1_w2, nn1_b2, root1, bias1,
           nn2_w1, nn2_b1, nn2_w2, nn2_b2, root2, bias2,
           ep_w1, ep_b1, ep_w2, ep_b2, ep_w3, ep_b3, ep_w4, ep_b4,
           ep_w5, ep_b5):
    f32 = jnp.float32
    src = edge_index[0]
    dst = edge_index[1]
    pad = EPAD - E
    src2 = jnp.pad(src, (0, pad)).reshape(ROWS, 128)
    dst2g = jnp.pad(dst, (0, pad)).reshape(ROWS, 128)
    dst2s = jnp.pad(dst, (0, pad), constant_values=N).reshape(ROWS, 128)
    srcq = [src2[q * QROWS:(q + 1) * QROWS] for q in range(NQ)]
    dstgq = [dst2g[q * QROWS:(q + 1) * QROWS] for q in range(NQ)]
    dstsq = [dst2s[q * QROWS:(q + 1) * QROWS] for q in range(NQ)]

    x_bn, se_c, te_c = _prep(x, e.T,
                             bn_node_g.reshape(1, -1), bn_node_b.reshape(1, -1),
                             bn_edge_g.reshape(-1, 1), bn_edge_b.reshape(-1, 1))
    se = se_c.reshape(1, -1)
    te = te_c.reshape(1, -1)

    zeros32 = jnp.zeros((NACC, 32), f32)
    layers = [(16, 32, nn0_w1, nn0_b1, nn0_w2, nn0_b2, root0, bias0),
              (32, 32, nn1_w1, nn1_b1, nn1_w2, nn1_b2, root1, bias1),
              (32, 32, nn2_w1, nn2_b1, nn2_w2, nn2_b2, root2, bias2)]
    xcur = x_bn
    for nin, nout, w1, b1, w2, b2, root, bias in layers:
        rm = jnp.repeat(jnp.eye(nin, dtype=f32), nout, axis=1)
        fm = jnp.tile(jnp.eye(nout, dtype=f32), (nin, 1))
        b1r = b1.reshape(1, -1)
        b2r = b2.reshape(1, -1)
        parts = []
        for q in range(NQ):
            (xg,) = _sc_gather(xcur, [srcq[q]], nin)
            msg = _msg(e, se, te, xg, w1, b1r, w2, b2r, rm, fm, nin, nout, q)
            parts.append(_sc_scatter(msg, dstsq[q], zeros32, nout))
        xcur = _update(parts, xcur, root, bias.reshape(1, -1))

    outs = []
    for q in range(NQ):
        xs, xd = _sc_gather(xcur, [srcq[q], dstgq[q]], 32)
        outs.append(_final(xs, xd, e, se, te,
                           ep_w1[0:32], ep_w1[32:64], ep_w1[64:74],
                           ep_b1.reshape(1, -1),
                           ep_w2, ep_b2.reshape(1, -1), ep_w3,
                           ep_b3.reshape(1, -1), ep_w4, ep_b4.reshape(1, -1),
                           ep_w5, ep_b5.reshape(1, -1), q))
    return jnp.concatenate(outs, axis=0)


# speedup vs baseline: 3.7814x; 1.0675x over previous
import functools

import jax
import jax.numpy as jnp
from jax import lax
from jax.experimental import pallas as pl
from jax.experimental.pallas import tpu as pltpu
from jax.experimental.pallas import tpu_sc as plsc

N = 10000
E = 160000
EPAD = 163840
ROWS = EPAD // 128
NQ = 4
QROWS = ROWS // NQ
QE = QROWS * 128
NACC = 10016
NC, NS = 2, 16
NW = NC * NS
RPQ = QROWS // NW
LEAK = 0.1
EPS = 1e-5
BLK = 1280


def _leaky(v):
    return jnp.where(v >= 0, v, LEAK * v)


def _dot(a, b):
    return lax.dot_general(a, b, (((1,), (0,)), ((), ())),
                           preferred_element_type=jnp.float32,
                           precision=lax.Precision.DEFAULT)


def _nblk(q):
    lo = q * QE
    hi = min((q + 1) * QE, E)
    assert (hi - lo) % BLK == 0
    return (hi - lo) // BLK


def _prep(x, et, gx, bx, ge, be):
    def body(x_ref, et_ref, gx_ref, bx_ref, ge_ref, be_ref,
             xo_ref, xo16_ref, se_ref, te_ref):
        xv = x_ref[...]
        m = jnp.mean(xv, axis=0, keepdims=True)
        v = jnp.mean((xv - m) ** 2, axis=0, keepdims=True)
        xbn = (xv - m) * lax.rsqrt(v + EPS) * gx_ref[...] + bx_ref[...]
        xo_ref[...] = xbn
        xo16_ref[:, 0:16] = xbn.astype(jnp.bfloat16)
        xo16_ref[:, 16:32] = jnp.zeros((N, 16), jnp.bfloat16)
        ev = et_ref[...]
        me = jnp.mean(ev, axis=1, keepdims=True)
        ve = jnp.mean((ev - me) ** 2, axis=1, keepdims=True)
        s = lax.rsqrt(ve + EPS) * ge_ref[...]
        se_ref[...] = s
        te_ref[...] = be_ref[...] - me * s

    return pl.pallas_call(
        body,
        out_shape=(jax.ShapeDtypeStruct((N, 16), jnp.float32),
                   jax.ShapeDtypeStruct((N, 32), jnp.bfloat16),
                   jax.ShapeDtypeStruct((10, 1), jnp.float32),
                   jax.ShapeDtypeStruct((10, 1), jnp.float32)),
    )(x, et, gx, bx, ge, be)


def _sc_gather(table, idx2s, d):
    n_out = len(idx2s)
    mesh = plsc.VectorSubcoreMesh(core_axis_name="c", subcore_axis_name="s")

    @functools.partial(
        pl.kernel,
        out_type=tuple(jax.ShapeDtypeStruct((QE, d), jnp.bfloat16)
                       for _ in range(n_out)),
        mesh=mesh,
        compiler_params=pltpu.CompilerParams(use_tc_tiling_on_sc=False),
        scratch_types=[pltpu.VMEM((RPQ, 128), jnp.int32),
                       pltpu.VMEM((128, d), jnp.bfloat16)],
    )
    def k(table_hbm, *refs):
        idx_hbms = refs[:n_out]
        out_hbms = refs[n_out:2 * n_out]
        idx_v, rows_v = refs[2 * n_out:]
        c = lax.axis_index("c")
        s = lax.axis_index("s")
        row0 = (s * NC + c) * RPQ
        for idx_hbm, out_hbm in zip(idx_hbms, out_hbms):
            pltpu.sync_copy(idx_hbm.at[pl.ds(row0, RPQ)], idx_v)

            @pl.loop(0, RPQ)
            def _row(r):
                pltpu.sync_copy(table_hbm.at[idx_v.at[r]], rows_v)
                pltpu.sync_copy(rows_v,
                                out_hbm.at[pl.ds((row0 + r) * 128, 128)])

    outs = k(table, *idx2s)
    return list(outs) if isinstance(outs, (tuple, list)) else [outs]


def _sc_scatter(msgq, dst2q, zeros, d):
    mesh = plsc.VectorSubcoreMesh(core_axis_name="c", subcore_axis_name="s")
    rps = NACC // NS

    @functools.partial(
        pl.kernel,
        out_type=jax.ShapeDtypeStruct((NC, NACC, d), jnp.float32),
        mesh=mesh,
        compiler_params=pltpu.CompilerParams(use_tc_tiling_on_sc=False),
        scratch_types=[pltpu.VMEM((RPQ, 128), jnp.int32),
                       pltpu.VMEM((128, d), jnp.float32),
                       pltpu.VMEM_SHARED((NACC, d), jnp.float32)],
    )
    def k(msg_hbm, dst_hbm, z_hbm, out_hbm, idx_v, rows_v, acc_sh):
        c = lax.axis_index("c")
        s = lax.axis_index("s")
        row0 = (s * NC + c) * RPQ
        rs = s * rps
        pltpu.sync_copy(z_hbm.at[pl.ds(rs, rps)], acc_sh.at[pl.ds(rs, rps)])
        pltpu.sync_copy(dst_hbm.at[pl.ds(row0, RPQ)], idx_v)
        plsc.subcore_barrier()

        @pl.loop(0, RPQ)
        def _row(r):
            pltpu.sync_copy(msg_hbm.at[pl.ds((row0 + r) * 128, 128)], rows_v)
            pltpu.sync_copy(rows_v, acc_sh.at[idx_v.at[r]], add=True)

        plsc.subcore_barrier()
        pltpu.sync_copy(acc_sh.at[pl.ds(rs, rps)], out_hbm.at[c, pl.ds(rs, rps)])

    return k(msgq, dst2q, zeros)


def _msg(e, se, te, xgq, w1, b1, w2, b2, rm, fm, nin, nout, q):
    g = _nblk(q)
    eoff = q * QE // BLK

    def body(e_ref, se_ref, te_ref, xg_ref, w1_ref, b1_ref, w2_ref, b2_ref,
             r_ref, f_ref, o_ref):
        eb = e_ref[...] * se_ref[...] + te_ref[...]
        h1 = _leaky(_dot(eb, w1_ref[...]) + b1_ref[...])
        w = _leaky(_dot(h1, w2_ref[...]) + b2_ref[...])
        rep = _dot(xg_ref[...], r_ref[...])
        o_ref[...] = _dot(w * rep, f_ref[...])

    k = nin * nout
    return pl.pallas_call(
        body,
        grid=(g,),
        in_specs=[pl.BlockSpec((BLK, 10), lambda i: (i + eoff, 0)),
                  pl.BlockSpec((1, 10), lambda i: (0, 0)),
                  pl.BlockSpec((1, 10), lambda i: (0, 0)),
                  pl.BlockSpec((BLK, 32), lambda i: (i, 0)),
                  pl.BlockSpec((10, nin), lambda i: (0, 0)),
                  pl.BlockSpec((1, nin), lambda i: (0, 0)),
                  pl.BlockSpec((nin, k), lambda i: (0, 0)),
                  pl.BlockSpec((1, k), lambda i: (0, 0)),
                  pl.BlockSpec((32, k), lambda i: (0, 0)),
                  pl.BlockSpec((k, nout), lambda i: (0, 0))],
        out_specs=pl.BlockSpec((BLK, nout), lambda i: (i, 0)),
        out_shape=jax.ShapeDtypeStruct((QE, nout), jnp.float32),
        compiler_params=pltpu.CompilerParams(
            dimension_semantics=("parallel",)),
    )(e, se, te, xgq, w1, b1, w2, b2, rm, fm)


def _update(parts, xin, root, bias):
    nout = root.shape[1]
    nparts = len(parts)

    def body(*refs):
        p_refs = refs[:nparts]
        x_ref, r_ref, b_ref, o_ref, o16_ref = refs[nparts:]
        acc = _dot(x_ref[...], r_ref[...]) + b_ref[...]
        for p in p_refs:
            acc = acc + p[0] + p[1]
        o_ref[...] = acc
        o16_ref[...] = acc.astype(jnp.bfloat16)

    pspec = pl.BlockSpec((NC, N, nout), lambda i: (0, 0, 0))
    return pl.pallas_call(
        body,
        grid=(1,),
        in_specs=[pspec] * nparts + [
            pl.BlockSpec(xin.shape, lambda i: (0, 0)),
            pl.BlockSpec(root.shape, lambda i: (0, 0)),
            pl.BlockSpec(bias.shape, lambda i: (0, 0))],
        out_specs=(pl.BlockSpec((N, nout), lambda i: (0, 0)),
                   pl.BlockSpec((N, nout), lambda i: (0, 0))),
        out_shape=(jax.ShapeDtypeStruct((N, nout), jnp.float32),
                   jax.ShapeDtypeStruct((N, nout), jnp.bfloat16)),
    )(*parts, xin, root, bias)


def _final(xsq, xdq, e, se, te, wa, wb, wc, b1, w2, b2, w3, b3, w4, b4,
           w5, b5, q):
    g = _nblk(q)
    eoff = q * QE // BLK

    def body(xs_ref, xd_ref, e_ref, se_ref, te_ref, wa_ref, wb_ref, wc_ref,
             b1_ref, w2_ref, b2_ref, w3_ref, b3_ref, w4_ref, b4_ref,
             w5_ref, b5_ref, o_ref):
        eb = e_ref[...] * se_ref[...] + te_ref[...]
        xs = xs_ref[...].astype(jnp.float32)
        xd = xd_ref[...].astype(jnp.float32)
        h = _leaky(_dot(xs, wa_ref[...])
                   + _dot(xd, wb_ref[...])
                   + _dot(eb, wc_ref[...]) + b1_ref[...])
        h = _leaky(_dot(h, w2_ref[...]) + b2_ref[...])
        h = _leaky(_dot(h, w3_ref[...]) + b3_ref[...])
        h = _leaky(_dot(h, w4_ref[...]) + b4_ref[...])
        o_ref[...] = _dot(h, w5_ref[...]) + b5_ref[...]

    return pl.pallas_call(
        body,
        grid=(g,),
        in_specs=[pl.BlockSpec((BLK, 32), lambda i: (i, 0)),
                  pl.BlockSpec((BLK, 32), lambda i: (i, 0)),
                  pl.BlockSpec((BLK, 10), lambda i: (i + eoff, 0)),
                  pl.BlockSpec((1, 10), lambda i: (0, 0)),
                  pl.BlockSpec((1, 10), lambda i: (0, 0)),
                  pl.BlockSpec((32, 64), lambda i: (0, 0)),
                  pl.BlockSpec((32, 64), lambda i: (0, 0)),
                  pl.BlockSpec((10, 64), lambda i: (0, 0)),
                  pl.BlockSpec((1, 64), lambda i: (0, 0)),
                  pl.BlockSpec((64, 32), lambda i: (0, 0)),
                  pl.BlockSpec((1, 32), lambda i: (0, 0)),
                  pl.BlockSpec((32, 16), lambda i: (0, 0)),
                  pl.BlockSpec((1, 16), lambda i: (0, 0)),
                  pl.BlockSpec((16, 8), lambda i: (0, 0)),
                  pl.BlockSpec((1, 8), lambda i: (0, 0)),
                  pl.BlockSpec((8, 2), lambda i: (0, 0)),
                  pl.BlockSpec((1, 2), lambda i: (0, 0))],
        out_specs=pl.BlockSpec((BLK, 2), lambda i: (i, 0)),
        out_shape=jax.ShapeDtypeStruct((g * BLK, 2), jnp.float32),
        compiler_params=pltpu.CompilerParams(
            dimension_semantics=("parallel",)),
    )(xsq, xdq, e, se, te, wa, wb, wc, b1, w2, b2, w3, b3, w4, b4, w5, b5)


def kernel(x, edge_index, e, xbatch, bn_node_g, bn_node_b, bn_edge_g, bn_edge_b,
           nn0_w1, nn0_b1, nn0_w2, nn0_b2, root0, bias0,
           nn1_w1, nn1_b1, nn1_w2, nn1_b2, root1, bias1,
           nn2_w1, nn2_b1, nn2_w2, nn2_b2, root2, bias2,
           ep_w1, ep_b1, ep_w2, ep_b2, ep_w3, ep_b3, ep_w4, ep_b4,
           ep_w5, ep_b5):
    f32 = jnp.float32
    src = edge_index[0]
    dst = edge_index[1]
    pad = EPAD - E
    src2 = jnp.pad(src, (0, pad)).reshape(ROWS, 128)
    dst2g = jnp.pad(dst, (0, pad)).reshape(ROWS, 128)
    dst2s = jnp.pad(dst, (0, pad), constant_values=N).reshape(ROWS, 128)
    srcq = [src2[q * QROWS:(q + 1) * QROWS] for q in range(NQ)]
    dstgq = [dst2g[q * QROWS:(q + 1) * QROWS] for q in range(NQ)]
    dstsq = [dst2s[q * QROWS:(q + 1) * QROWS] for q in range(NQ)]

    x_bn, x16, se_c, te_c = _prep(x, e.T,
                                  bn_node_g.reshape(1, -1),
                                  bn_node_b.reshape(1, -1),
                                  bn_edge_g.reshape(-1, 1),
                                  bn_edge_b.reshape(-1, 1))
    se = se_c.reshape(1, -1)
    te = te_c.reshape(1, -1)

    zeros32 = jnp.zeros((NACC, 32), f32)
    layers = [(16, 32, nn0_w1, nn0_b1, nn0_w2, nn0_b2, root0, bias0),
              (32, 32, nn1_w1, nn1_b1, nn1_w2, nn1_b2, root1, bias1),
              (32, 32, nn2_w1, nn2_b1, nn2_w2, nn2_b2, root2, bias2)]
    xcur, x16cur = x_bn, x16
    for nin, nout, w1, b1, w2, b2, root, bias in layers:
        rm = jnp.zeros((32, nin * nout), jnp.bfloat16).at[:nin].set(
            jnp.repeat(jnp.eye(nin, dtype=jnp.bfloat16), nout, axis=1))
        fm = jnp.tile(jnp.eye(nout, dtype=f32), (nin, 1))
        b1r = b1.reshape(1, -1)
        b2r = b2.reshape(1, -1)
        parts = []
        for q in range(NQ):
            (xg,) = _sc_gather(x16cur, [srcq[q]], 32)
            msg = _msg(e, se, te, xg, w1, b1r, w2, b2r, rm, fm, nin, nout, q)
            parts.append(_sc_scatter(msg, dstsq[q], zeros32, nout))
        xcur, x16cur = _update(parts, xcur, root, bias.reshape(1, -1))

    outs = []
    for q in range(NQ):
        xs, xd = _sc_gather(x16cur, [srcq[q], dstgq[q]], 32)
        outs.append(_final(xs, xd, e, se, te,
                           ep_w1[0:32], ep_w1[32:64], ep_w1[64:74],
                           ep_b1.reshape(1, -1),
                           ep_w2, ep_b2.reshape(1, -1), ep_w3,
                           ep_b3.reshape(1, -1), ep_w4, ep_b4.reshape(1, -1),
                           ep_w5, ep_b5.reshape(1, -1), q))
    return jnp.concatenate(outs, axis=0)
